# Initial kernel scaffold; baseline (speedup 1.0000x reference)
import jax
import jax.numpy as jnp
from jax.experimental import pallas as pl


def kernel(x, edge_index, W1, b1, W2, b2):
    raise NotImplementedError("write your pallas kernel here")


# speedup vs baseline: 16.6131x; 16.6131x over previous
import functools

import jax
import jax.numpy as jnp
from jax import lax
from jax.experimental import pallas as pl
from jax.experimental.pallas import tpu as pltpu
from jax.experimental.pallas import tpu_sc as plsc

_SC_PARAMS = pltpu.CompilerParams(use_tc_tiling_on_sc=False)

NC = 2
NS = 16
NW = NC * NS
CHUNK = 128
DEGW = 16


def _sc_degree(dst_r, ones_blk, zeros_blk, n_acc, rpt):
    n_chunks = dst_r.shape[1]
    mesh = plsc.VectorSubcoreMesh(core_axis_name="c", subcore_axis_name="s")

    @functools.partial(
        pl.kernel,
        mesh=mesh,
        out_type=jax.ShapeDtypeStruct((NC, n_acc, DEGW), jnp.float32),
        compiler_params=_SC_PARAMS,
        scratch_types=[
            pltpu.VMEM((n_chunks, CHUNK), jnp.int32),
            pltpu.VMEM((CHUNK, DEGW), jnp.float32),
            pltpu.VMEM_SHARED((n_acc, DEGW), jnp.float32),
        ],
    )
    def deg_kernel(dst_hbm, ones_hbm, zeros_hbm, out_hbm, dst_v, ones_v, acc_sh):
        c = lax.axis_index("c")
        s = lax.axis_index("s")
        wid = s * NC + c
        pltpu.sync_copy(zeros_hbm, acc_sh.at[pl.ds(s * rpt, rpt)])
        pltpu.sync_copy(dst_hbm.at[wid], dst_v)
        pltpu.sync_copy(ones_hbm, ones_v)
        plsc.subcore_barrier()

        @pl.loop(0, n_chunks)
        def _(j):
            pltpu.sync_copy(ones_v, acc_sh.at[dst_v.at[j]], add=True)

        plsc.subcore_barrier()
        pltpu.sync_copy(acc_sh.at[pl.ds(s * rpt, rpt)],
                        out_hbm.at[c].at[pl.ds(s * rpt, rpt)])

    return deg_kernel(dst_r, ones_blk, zeros_blk)


def _sc_edge_pass(hs, src_r, dst_r, zeros_blk, n_acc, rpt):
    n_chunks = src_r.shape[1]
    d = hs.shape[1]
    mesh = plsc.VectorSubcoreMesh(core_axis_name="c", subcore_axis_name="s")

    @functools.partial(
        pl.kernel,
        mesh=mesh,
        out_type=jax.ShapeDtypeStruct((NC, n_acc, d), jnp.float32),
        compiler_params=_SC_PARAMS,
        scratch_types=[
            pltpu.VMEM((n_chunks, CHUNK), jnp.int32),
            pltpu.VMEM((n_chunks, CHUNK), jnp.int32),
            pltpu.VMEM((CHUNK, d), jnp.float32),
            pltpu.VMEM_SHARED((n_acc, d), jnp.float32),
            pltpu.SemaphoreType.DMA,
        ],
    )
    def edge_kernel(hs_hbm, src_hbm, dst_hbm, zeros_hbm, out_hbm,
                    src_v, dst_v, rows_v, acc_sh, sem):
        c = lax.axis_index("c")
        s = lax.axis_index("s")
        wid = s * NC + c
        pltpu.sync_copy(zeros_hbm, acc_sh.at[pl.ds(s * rpt, rpt)])
        pltpu.sync_copy(src_hbm.at[wid], src_v)
        pltpu.sync_copy(dst_hbm.at[wid], dst_v)
        plsc.subcore_barrier()

        @pl.loop(0, n_chunks)
        def _(j):
            pltpu.async_copy(hs_hbm.at[src_v.at[j]], rows_v, sem).wait()
            pltpu.sync_copy(rows_v, acc_sh.at[dst_v.at[j]], add=True)

        plsc.subcore_barrier()
        pltpu.sync_copy(acc_sh.at[pl.ds(s * rpt, rpt)],
                        out_hbm.at[c].at[pl.ds(s * rpt, rpt)])

    return edge_kernel(hs, src_r, dst_r, zeros_blk)


def _tc_mm_scale(x, w, degp, blk):
    n, d = x.shape
    h = w.shape[1]

    def body(x_ref, w_ref, deg_ref, o_ref):
        cnt = deg_ref[0, :, 0:1] + deg_ref[1, :, 0:1]
        dinv = lax.rsqrt(cnt + 1.0)
        o_ref[...] = jnp.dot(x_ref[...], w_ref[...],
                             preferred_element_type=jnp.float32) * dinv

    return pl.pallas_call(
        body,
        grid=(n // blk,),
        in_specs=[
            pl.BlockSpec((blk, d), lambda i: (i, 0)),
            pl.BlockSpec((d, h), lambda i: (0, 0)),
            pl.BlockSpec((2, blk, DEGW), lambda i: (0, i, 0)),
        ],
        out_specs=pl.BlockSpec((blk, h), lambda i: (i, 0)),
        out_shape=jax.ShapeDtypeStruct((n, h), jnp.float32),
    )(x, w, degp)


def _tc_mid(p, hs1, degp, b1, w2, blk):
    n, h = hs1.shape
    c_out = w2.shape[1]

    def body(p_ref, hs1_ref, deg_ref, b1_ref, w2_ref, o_ref):
        cnt = deg_ref[0, :, 0:1] + deg_ref[1, :, 0:1]
        dinv = lax.rsqrt(cnt + 1.0)
        a = (p_ref[0] + p_ref[1] + hs1_ref[...]) * dinv + b1_ref[...]
        r = jnp.maximum(a, 0.0)
        o_ref[...] = jnp.dot(r, w2_ref[...],
                             preferred_element_type=jnp.float32) * dinv

    return pl.pallas_call(
        body,
        grid=(n // blk,),
        in_specs=[
            pl.BlockSpec((2, blk, h), lambda i: (0, i, 0)),
            pl.BlockSpec((blk, h), lambda i: (i, 0)),
            pl.BlockSpec((2, blk, DEGW), lambda i: (0, i, 0)),
            pl.BlockSpec((1, h), lambda i: (0, 0)),
            pl.BlockSpec((h, c_out), lambda i: (0, 0)),
        ],
        out_specs=pl.BlockSpec((blk, c_out), lambda i: (i, 0)),
        out_shape=jax.ShapeDtypeStruct((n, c_out), jnp.float32),
    )(p, hs1, degp, b1, w2)


def _tc_post(q, hs2, degp, b2, blk):
    n, c_out = hs2.shape

    def body(q_ref, hs2_ref, deg_ref, b2_ref, o_ref):
        cnt = deg_ref[0, :, 0:1] + deg_ref[1, :, 0:1]
        dinv = lax.rsqrt(cnt + 1.0)
        a = (q_ref[0] + q_ref[1] + hs2_ref[...]) * dinv + b2_ref[...]
        m = jnp.max(a, axis=1, keepdims=True)
        lse = m + jnp.log(jnp.sum(jnp.exp(a - m), axis=1, keepdims=True))
        o_ref[...] = a - lse

    return pl.pallas_call(
        body,
        grid=(n // blk,),
        in_specs=[
            pl.BlockSpec((2, blk, c_out), lambda i: (0, i, 0)),
            pl.BlockSpec((blk, c_out), lambda i: (i, 0)),
            pl.BlockSpec((2, blk, DEGW), lambda i: (0, i, 0)),
            pl.BlockSpec((1, c_out), lambda i: (0, 0)),
        ],
        out_specs=pl.BlockSpec((blk, c_out), lambda i: (i, 0)),
        out_shape=jax.ShapeDtypeStruct((n, c_out), jnp.float32),
    )(q, hs2, degp, b2)


def kernel(x, edge_index, W1, b1, W2, b2):
    n, d = x.shape
    h = W1.shape[1]
    c_out = W2.shape[1]
    e = edge_index.shape[1]

    n_chunks = -(-e // (NW * CHUNK))
    e_pad = NW * n_chunks * CHUNK
    src = jnp.concatenate(
        [edge_index[0], jnp.zeros((e_pad - e,), jnp.int32)])
    dst = jnp.concatenate(
        [edge_index[1], jnp.full((e_pad - e,), n, jnp.int32)])
    src_r = src.reshape(NW, n_chunks, CHUNK)
    dst_r = dst.reshape(NW, n_chunks, CHUNK)

    rpt = 8 * (-(-(n + 1) // (NS * 8)))
    n_acc = rpt * NS

    zeros_deg = jnp.zeros((rpt, DEGW), jnp.float32)
    zeros_h = jnp.zeros((rpt, h), jnp.float32)
    zeros_c = jnp.zeros((rpt, c_out), jnp.float32)
    ones_blk = jnp.ones((CHUNK, DEGW), jnp.float32)

    blk = 1000 if n % 1000 == 0 else 8 * (-(-n // 8))

    degp = _sc_degree(dst_r, ones_blk, zeros_deg, n_acc, rpt)
    hs1 = _tc_mm_scale(x, W1, degp, blk)
    p = _sc_edge_pass(hs1, src_r, dst_r, zeros_h, n_acc, rpt)
    hs2 = _tc_mid(p, hs1, degp, b1.reshape(1, h), W2, blk)
    q = _sc_edge_pass(hs2, src_r, dst_r, zeros_c, n_acc, rpt)
    return _tc_post(q, hs2, degp, b2.reshape(1, c_out), blk)
